# baseline (speedup 1.0000x reference)
import functools

import jax
import jax.numpy as jnp
from jax import lax
from jax.experimental import pallas as pl
from jax.experimental.pallas import tpu as pltpu
from jax.experimental.pallas import tpu_sc as plsc

EMBED = 64
SEQ = 512
EPS = 1e-5
NW = 32
CHUNK = 256
BB = 8


def _make_gather(n_rows):
    rows_per_w = n_rows // NW

    mesh = plsc.VectorSubcoreMesh(core_axis_name="c", subcore_axis_name="s")

    @functools.partial(
        pl.kernel,
        mesh=mesh,
        compiler_params=pltpu.CompilerParams(use_tc_tiling_on_sc=True),
        out_type=jax.ShapeDtypeStruct((n_rows, 2 * EMBED), jnp.float32),
        scratch_types=[
            pltpu.VMEM((CHUNK,), jnp.int32),
            pltpu.VMEM((CHUNK,), jnp.int32),
            pltpu.VMEM((CHUNK, 2 * EMBED), jnp.float32),
            pltpu.VMEM((CHUNK, 2 * EMBED), jnp.float32),
            pltpu.SemaphoreType.DMA,
            pltpu.SemaphoreType.DMA,
        ],
    )
    def gather(ids_hbm, tok_hbm, out_hbm, idx0, idx1, rows0, rows1,
               sem0, sem1):
        wid = lax.axis_index("s") * 2 + lax.axis_index("c")
        base = wid * rows_per_w

        def body(i, _):
            off0 = base + i * (2 * CHUNK)
            off1 = off0 + CHUNK
            pltpu.sync_copy(ids_hbm.at[pl.ds(off0, CHUNK)], idx0)
            h0 = pltpu.async_copy(tok_hbm.at[idx0], rows0, sem0)
            pltpu.sync_copy(ids_hbm.at[pl.ds(off1, CHUNK)], idx1)
            h1 = pltpu.async_copy(tok_hbm.at[idx1], rows1, sem1)
            h0.wait()
            pltpu.sync_copy(rows0, out_hbm.at[pl.ds(off0, CHUNK)])
            h1.wait()
            pltpu.sync_copy(rows1, out_hbm.at[pl.ds(off1, CHUNK)])
            return 0

        lax.fori_loop(0, rows_per_w // (2 * CHUNK), body, 0)

    return gather


def _ln_body(x2_ref, ids_ref, pos2_ref, g_ref, b_ref, o_ref):
    nc = SEQ // 128
    x2 = x2_ref[...].reshape(-1, nc, 128, 128)
    xp = x2 + pos2_ref[...].reshape(1, nc, 128, 128)
    xt = jnp.swapaxes(xp, 2, 3)
    lo = xt[:, :, :EMBED, :]
    hi = xt[:, :, EMBED:, :]
    odd = (ids_ref[...].reshape(-1, nc, 1, 128) & 1) == 1
    x = jnp.where(odd, hi, lo)
    s1 = jnp.sum(x, axis=2, keepdims=True)
    s2 = jnp.sum(x * x, axis=2, keepdims=True)
    mean = s1 * (1.0 / EMBED)
    var = s2 * (1.0 / EMBED) - mean * mean
    inv = lax.rsqrt(var + EPS)
    g = g_ref[...].reshape(1, 1, EMBED, 1)
    b = b_ref[...].reshape(1, 1, EMBED, 1)
    y = (x - mean) * inv * g + b
    yt = jnp.swapaxes(y, 1, 2)
    o_ref[...] = yt.reshape(yt.shape[0], EMBED, SEQ)


def kernel(input_ids, token_table, pos_table, gamma, beta):
    batch, seq = input_ids.shape
    n_rows = batch * seq
    ids_half = input_ids.reshape(n_rows) >> 1
    tok2 = token_table.reshape(-1, 2 * EMBED)
    pos2 = jnp.concatenate([pos_table, pos_table], axis=1)

    nh = batch // 2
    gather = _make_gather(n_rows // 2)
    ln = functools.partial(
        pl.pallas_call,
        _ln_body,
        grid=(nh // BB,),
        in_specs=[
            pl.BlockSpec((BB * seq, 2 * EMBED), lambda i: (i, 0)),
            pl.BlockSpec((BB, seq), lambda i: (i, 0)),
            pl.BlockSpec((seq, 2 * EMBED), lambda i: (0, 0)),
            pl.BlockSpec((EMBED, 1), lambda i: (0, 0)),
            pl.BlockSpec((EMBED, 1), lambda i: (0, 0)),
        ],
        out_specs=pl.BlockSpec((BB, EMBED, seq), lambda i: (i, 0, 0)),
        out_shape=jax.ShapeDtypeStruct((nh, EMBED, seq), jnp.float32),
    )
    g_a = gather(ids_half[: n_rows // 2], tok2)
    g_b = gather(ids_half[n_rows // 2 :], tok2)
    ga = gamma.reshape(EMBED, 1)
    be = beta.reshape(EMBED, 1)
    out_a = ln()(g_a, input_ids[:nh], pos2, ga, be)
    out_b = ln()(g_b, input_ids[nh:], pos2, ga, be)
    out_t = jnp.concatenate([out_a, out_b], axis=0)
    return jnp.transpose(out_t, (0, 2, 1))


# speedup vs baseline: 1.0370x; 1.0370x over previous
import functools

import jax
import jax.numpy as jnp
from jax import lax
from jax.experimental import pallas as pl
from jax.experimental.pallas import tpu as pltpu
from jax.experimental.pallas import tpu_sc as plsc

EMBED = 64
SEQ = 512
EPS = 1e-5
NW = 32
CHUNK = 256
BB = 8


def _make_gather(n_rows):
    rows_per_w = n_rows // NW

    mesh = plsc.VectorSubcoreMesh(core_axis_name="c", subcore_axis_name="s")

    @functools.partial(
        pl.kernel,
        mesh=mesh,
        compiler_params=pltpu.CompilerParams(use_tc_tiling_on_sc=True),
        out_type=jax.ShapeDtypeStruct((n_rows, 2 * EMBED), jnp.float32),
        scratch_types=[
            pltpu.VMEM((CHUNK,), jnp.int32),
            pltpu.VMEM((CHUNK,), jnp.int32),
            pltpu.VMEM((CHUNK, 2 * EMBED), jnp.float32),
            pltpu.VMEM((CHUNK, 2 * EMBED), jnp.float32),
            pltpu.SemaphoreType.DMA,
            pltpu.SemaphoreType.DMA,
        ],
    )
    def gather(ids_hbm, tok_hbm, out_hbm, idx0, idx1, rows0, rows1,
               sem0, sem1):
        wid = lax.axis_index("s") * 2 + lax.axis_index("c")
        base = wid * rows_per_w

        def body(i, _):
            off0 = base + i * (2 * CHUNK)
            off1 = off0 + CHUNK
            pltpu.sync_copy(ids_hbm.at[pl.ds(off0, CHUNK)], idx0)
            h0 = pltpu.async_copy(tok_hbm.at[idx0], rows0, sem0)
            pltpu.sync_copy(ids_hbm.at[pl.ds(off1, CHUNK)], idx1)
            h1 = pltpu.async_copy(tok_hbm.at[idx1], rows1, sem1)
            h0.wait()
            pltpu.sync_copy(rows0, out_hbm.at[pl.ds(off0, CHUNK)])
            h1.wait()
            pltpu.sync_copy(rows1, out_hbm.at[pl.ds(off1, CHUNK)])
            return 0

        lax.fori_loop(0, rows_per_w // (2 * CHUNK), body, 0)

    return gather


def _ln_body(x2_ref, ids_ref, pos2_ref, g_ref, b_ref, o_ref):
    nc = SEQ // 128
    x2 = x2_ref[...].reshape(-1, nc, 128, 128)
    xp = x2 + pos2_ref[...].reshape(1, nc, 128, 128)
    xt = jnp.swapaxes(xp, 2, 3)
    lo = xt[:, :, :EMBED, :]
    hi = xt[:, :, EMBED:, :]
    odd = (ids_ref[...].reshape(-1, nc, 1, 128) & 1) == 1
    x = jnp.where(odd, hi, lo)
    s1 = jnp.sum(x, axis=2, keepdims=True)
    s2 = jnp.sum(x * x, axis=2, keepdims=True)
    mean = s1 * (1.0 / EMBED)
    var = s2 * (1.0 / EMBED) - mean * mean
    inv = lax.rsqrt(var + EPS)
    g = g_ref[...].reshape(1, 1, EMBED, 1)
    b = b_ref[...].reshape(1, 1, EMBED, 1)
    y = (x - mean) * inv * g + b
    yt = jnp.swapaxes(y, 1, 2)
    o_ref[...] = yt.reshape(yt.shape[0], EMBED, SEQ)


def kernel(input_ids, token_table, pos_table, gamma, beta):
    batch, seq = input_ids.shape
    n_rows = batch * seq
    ids_half = input_ids.reshape(n_rows) >> 1
    tok2 = token_table.reshape(-1, 2 * EMBED)
    pos2 = jnp.concatenate([pos_table, pos_table], axis=1)

    g2 = _make_gather(n_rows)(ids_half, tok2)

    out_t = pl.pallas_call(
        _ln_body,
        grid=(batch // BB,),
        in_specs=[
            pl.BlockSpec((BB * seq, 2 * EMBED), lambda i: (i, 0)),
            pl.BlockSpec((BB, seq), lambda i: (i, 0)),
            pl.BlockSpec((seq, 2 * EMBED), lambda i: (0, 0)),
            pl.BlockSpec((EMBED, 1), lambda i: (0, 0)),
            pl.BlockSpec((EMBED, 1), lambda i: (0, 0)),
        ],
        out_specs=pl.BlockSpec((BB, EMBED, seq), lambda i: (i, 0, 0)),
        out_shape=jax.ShapeDtypeStruct((batch, EMBED, seq), jnp.float32),
    )(g2, input_ids, pos2, gamma.reshape(EMBED, 1), beta.reshape(EMBED, 1))
    return jnp.transpose(out_t, (0, 2, 1))
